# baseline (speedup 1.0000x reference)
import functools

import jax
import jax.numpy as jnp
from jax.experimental import pallas as pl
from jax.experimental.pallas import tpu as pltpu


def _prep_kernel(x_ref, pscale_ref, pbias_ref, xp_ref):
    xp_ref[...] = (x_ref[...] * pscale_ref[...] + pbias_ref[...]
                   ).astype(jnp.bfloat16)


def _fused_sae_kernel(xp_ref, enc_ref, dec_ref, lb_ref, qscale_ref, qbias_ref,
                      z_ref, y_ref, zp_ref, *, n_blocks):
    nj = pl.program_id(1)
    bn = enc_ref.shape[1]

    @pl.when(nj == 0)
    def _init():
        y_ref[...] = jnp.zeros_like(y_ref)

    @pl.when((nj >= 2) & (nj % 2 == 0))
    def _decode():
        y_ref[...] += jnp.dot(zp_ref[...], dec_ref[...].astype(jnp.bfloat16),
                              preferred_element_type=jnp.float32)

    @pl.when(nj < n_blocks)
    def _encode():
        z = jnp.dot(xp_ref[...], enc_ref[...].astype(jnp.bfloat16),
                    preferred_element_type=jnp.float32)
        z = jnp.maximum(z + lb_ref[...], 0.0)
        z_ref[...] = z
        zb = z.astype(jnp.bfloat16)

        @pl.when(nj % 2 == 0)
        def _slot0():
            zp_ref[:, :bn] = zb

        @pl.when(nj % 2 == 1)
        def _slot1():
            zp_ref[:, bn:] = zb

    @pl.when(nj == n_blocks)
    def _finish():
        y_ref[...] = y_ref[...] * qscale_ref[...] + qbias_ref[...]


def kernel(x, encoder, decoder, pre_bias, latent_bias, mean_center, scaling_factor):
    m, d = x.shape
    n = encoder.shape[1]
    bm = min(2048, m)
    bn = min(512, n)
    m_blocks = m // bm
    n_blocks = n // bn

    s = scaling_factor.astype(jnp.float32)
    pscale = jnp.broadcast_to(s, (1, d))
    pbias = (-(mean_center * s + pre_bias)).reshape(1, d)
    qscale = jnp.broadcast_to(1.0 / s, (1, d))
    qbias = (pre_bias / s + mean_center).reshape(1, d)
    lb = latent_bias.reshape(1, n)

    xp = pl.pallas_call(
        _prep_kernel,
        grid=(4,),
        in_specs=[
            pl.BlockSpec((m // 4, d), lambda i: (i, 0)),
            pl.BlockSpec((1, d), lambda i: (0, 0)),
            pl.BlockSpec((1, d), lambda i: (0, 0)),
        ],
        out_specs=pl.BlockSpec((m // 4, d), lambda i: (i, 0)),
        out_shape=jax.ShapeDtypeStruct((m, d), jnp.bfloat16),
    )(x, pscale, pbias)

    last = n_blocks - 1
    pair_last = n_blocks // 2 - 1
    grid = (m_blocks, n_blocks + 1)
    kfn = functools.partial(_fused_sae_kernel, n_blocks=n_blocks)
    z, y = pl.pallas_call(
        kfn,
        grid=grid,
        in_specs=[
            pl.BlockSpec((bm, d), lambda i, j: (i, 0)),
            pl.BlockSpec((d, bn), lambda i, j: (0, jnp.minimum(j, last))),
            pl.BlockSpec((2 * bn, d),
                         lambda i, j: (jnp.clip(j // 2 - 1, 0, pair_last), 0)),
            pl.BlockSpec((1, bn), lambda i, j: (0, jnp.minimum(j, last))),
            pl.BlockSpec((1, d), lambda i, j: (0, 0)),
            pl.BlockSpec((1, d), lambda i, j: (0, 0)),
        ],
        out_specs=[
            pl.BlockSpec((bm, bn), lambda i, j: (i, jnp.minimum(j, last))),
            pl.BlockSpec((bm, d), lambda i, j: (i, 0)),
        ],
        out_shape=[
            jax.ShapeDtypeStruct((m, n), jnp.float32),
            jax.ShapeDtypeStruct((m, d), jnp.float32),
        ],
        scratch_shapes=[pltpu.VMEM((bm, 2 * bn), jnp.bfloat16)],
        compiler_params=pltpu.CompilerParams(
            dimension_semantics=("parallel", "arbitrary"),
        ),
    )(xp, encoder, decoder, lb, qscale, qbias)
    return (y, z)


# speedup vs baseline: 1.2419x; 1.2419x over previous
import functools

import jax
import jax.numpy as jnp
from jax.experimental import pallas as pl
from jax.experimental.pallas import tpu as pltpu


def _fused_sae_kernel(x_ref, enc_ref, dec_ref, lb_ref, pscale_ref, pbias_ref,
                      qscale_ref, qbias_ref, z_ref, y_ref, xp_ref, *, n_blocks):
    nj = pl.program_id(1)

    @pl.when(nj == 0)
    def _init():
        xp_ref[...] = (x_ref[...] * pscale_ref[...] + pbias_ref[...]
                       ).astype(jnp.bfloat16)
        y_ref[...] = jnp.zeros_like(y_ref)

    z = jnp.dot(xp_ref[...], enc_ref[...].astype(jnp.bfloat16),
                preferred_element_type=jnp.float32)
    z = jnp.maximum(z + lb_ref[...], 0.0)
    z_ref[...] = z
    y_ref[...] += jnp.dot(z.astype(jnp.bfloat16),
                          dec_ref[...].astype(jnp.bfloat16),
                          preferred_element_type=jnp.float32)

    @pl.when(nj == n_blocks - 1)
    def _finish():
        y_ref[...] = y_ref[...] * qscale_ref[...] + qbias_ref[...]


def kernel(x, encoder, decoder, pre_bias, latent_bias, mean_center, scaling_factor):
    m, d = x.shape
    n = encoder.shape[1]
    bm = min(2048, m)
    bn = min(512, n)
    m_blocks = m // bm
    n_blocks = n // bn

    s = scaling_factor.astype(jnp.float32)
    pscale = jnp.broadcast_to(s, (1, d))
    pbias = (-(mean_center * s + pre_bias)).reshape(1, d)
    qscale = jnp.broadcast_to(1.0 / s, (1, d))
    qbias = (pre_bias / s + mean_center).reshape(1, d)
    lb = latent_bias.reshape(1, n)

    grid = (m_blocks, n_blocks)
    kfn = functools.partial(_fused_sae_kernel, n_blocks=n_blocks)
    z, y = pl.pallas_call(
        kfn,
        grid=grid,
        in_specs=[
            pl.BlockSpec((bm, d), lambda i, j: (i, 0)),
            pl.BlockSpec((d, bn), lambda i, j: (0, j)),
            pl.BlockSpec((bn, d), lambda i, j: (j, 0)),
            pl.BlockSpec((1, bn), lambda i, j: (0, j)),
            pl.BlockSpec((1, d), lambda i, j: (0, 0)),
            pl.BlockSpec((1, d), lambda i, j: (0, 0)),
            pl.BlockSpec((1, d), lambda i, j: (0, 0)),
            pl.BlockSpec((1, d), lambda i, j: (0, 0)),
        ],
        out_specs=[
            pl.BlockSpec((bm, bn), lambda i, j: (i, j)),
            pl.BlockSpec((bm, d), lambda i, j: (i, 0)),
        ],
        out_shape=[
            jax.ShapeDtypeStruct((m, n), jnp.float32),
            jax.ShapeDtypeStruct((m, d), jnp.float32),
        ],
        scratch_shapes=[pltpu.VMEM((bm, d), jnp.bfloat16)],
        compiler_params=pltpu.CompilerParams(
            dimension_semantics=("parallel", "arbitrary"),
        ),
    )(x, encoder, decoder, lb, pscale, pbias, qscale, qbias)
    return (y, z)
